# Initial kernel scaffold; baseline (speedup 1.0000x reference)
import jax
import jax.numpy as jnp
from jax.experimental import pallas as pl


def kernel(x, W1, b1, W2, b2):
    raise NotImplementedError("write your pallas kernel here")


# speedup vs baseline: 3.5465x; 3.5465x over previous
import functools

import jax
import jax.numpy as jnp
from jax.experimental import pallas as pl
from jax.experimental.pallas import tpu as pltpu

K = 10
_NEG_INF = float("-inf")


def _layer_body(x_ref, w_ref, res_ref, out_ref, pre_ref, *, nblk):
    x = x_ref[0]
    C, N = x.shape
    bs = N // nblk
    sq = jnp.sum(x * x, axis=0, keepdims=True)
    x_bf = x.astype(jnp.bfloat16)
    w_bf = w_ref[...].astype(jnp.bfloat16)
    for r in range(nblk):
        xb = x[:, r * bs:(r + 1) * bs]
        g = jax.lax.dot_general(xb.astype(jnp.bfloat16), x_bf,
                                (((0,), (0,)), ((), ())),
                                preferred_element_type=jnp.float32)
        d = 2.0 * g - sq
        iot = jax.lax.broadcasted_iota(jnp.int32, (bs, N), 1)
        xc_bf = xb.astype(jnp.bfloat16)
        acc = jnp.zeros((w_bf.shape[0], bs), jnp.float32)
        for _ in range(K):
            mx = jnp.max(d, axis=1, keepdims=True)
            eq = d == mx
            c = jnp.min(jnp.where(eq, iot, N), axis=1, keepdims=True)
            oh = (iot == c).astype(jnp.float32)
            d = jnp.where(oh != 0.0, _NEG_INF, d)
            nbr = jax.lax.dot_general(x, oh, (((1,), (1,)), ((), ())),
                                      precision=jax.lax.Precision.HIGHEST,
                                      preferred_element_type=jnp.float32)
            ff = jnp.concatenate(
                [(nbr - xb).astype(jnp.bfloat16), xc_bf], axis=0)
            acc = acc + jnp.dot(w_bf, ff, preferred_element_type=jnp.float32)
        pre_ref[:, r * bs:(r + 1) * bs] = acc / float(K)
    p = pre_ref[...]
    mu = jnp.mean(p, axis=1, keepdims=True)
    dev = p - mu
    var = jnp.sum(dev * dev, axis=1, keepdims=True) / (N - 1)
    y = dev / jnp.sqrt(var + 0.001)
    y = jnp.maximum(y, 0.0)
    if res_ref is not None:
        y = y + res_ref[0]
    out_ref[0] = y


def _layer(x, w, residual, *, nblk=8, interpret=False):
    B, C, N = x.shape
    O = w.shape[0]
    body = functools.partial(_layer_body, nblk=nblk)
    if residual is None:
        def kern(x_ref, w_ref, out_ref, pre_ref):
            body(x_ref, w_ref, None, out_ref, pre_ref)
        operands = (x, w)
        in_specs = [
            pl.BlockSpec((1, C, N), lambda b: (b, 0, 0)),
            pl.BlockSpec((O, 2 * C), lambda b: (0, 0)),
        ]
    else:
        kern = body
        operands = (x, w, residual)
        in_specs = [
            pl.BlockSpec((1, C, N), lambda b: (b, 0, 0)),
            pl.BlockSpec((O, 2 * C), lambda b: (0, 0)),
            pl.BlockSpec((1, O, N), lambda b: (b, 0, 0)),
        ]
    return pl.pallas_call(
        kern,
        grid=(B,),
        in_specs=in_specs,
        out_specs=pl.BlockSpec((1, O, N), lambda b: (b, 0, 0)),
        out_shape=jax.ShapeDtypeStruct((B, O, N), jnp.float32),
        scratch_shapes=[pltpu.VMEM((O, N), jnp.float32)],
        interpret=interpret,
    )(*operands)


def _forward(x, W1, b1, W2, b2, interpret=False):
    del b1, b2
    h = _layer(x, W1, None, interpret=interpret)
    return _layer(h, W2, x, interpret=interpret)


def kernel(x, W1, b1, W2, b2):
    return _forward(x, W1, b1, W2, b2)


# speedup vs baseline: 6.9168x; 1.9503x over previous
import functools

import jax
import jax.numpy as jnp
from jax import lax
from jax.experimental import pallas as pl
from jax.experimental.pallas import tpu as pltpu
from jax.experimental.pallas import tpu_sc as plsc

K = 10
KPAD = 16
_NEG_INF = float("-inf")

_SC_CORES = 2
_SC_SUBCORES = 16
_SC_WORKERS = _SC_CORES * _SC_SUBCORES
_GCHUNK = 128


def _topk_body(x_ref, idx_ref, *, nblk):
    x = x_ref[0]
    C, N = x.shape
    bs = N // nblk
    base = pl.program_id(0) * N
    sq = jnp.sum(x * x, axis=0, keepdims=True)
    x_bf = x.astype(jnp.bfloat16)
    for r in range(nblk):
        xb = x[:, r * bs:(r + 1) * bs]
        g = lax.dot_general(xb.astype(jnp.bfloat16), x_bf,
                            (((0,), (0,)), ((), ())),
                            preferred_element_type=jnp.float32)
        d = 2.0 * g - sq
        iot = lax.broadcasted_iota(jnp.int32, (bs, N), 1)
        cs = []
        for _ in range(K):
            mx = jnp.max(d, axis=1, keepdims=True)
            eq = d == mx
            c = jnp.min(jnp.where(eq, iot, N), axis=1, keepdims=True)
            cs.append(c)
            d = jnp.where(iot == c, _NEG_INF, d)
        idxmat = jnp.concatenate(cs + [cs[0]] * (KPAD - K), axis=1)
        idx_ref[0, r * bs:(r + 1) * bs, :] = idxmat + base


def _topk(x, *, nblk=8):
    B, C, N = x.shape
    return pl.pallas_call(
        functools.partial(_topk_body, nblk=nblk),
        grid=(B,),
        in_specs=[pl.BlockSpec((1, C, N), lambda b: (b, 0, 0))],
        out_specs=pl.BlockSpec((1, N, KPAD), lambda b: (b, 0, 0)),
        out_shape=jax.ShapeDtypeStruct((B, N, KPAD), jnp.int32),
    )(x)


def _sc_gather(table, idx):
    R = idx.shape[0]
    C = table.shape[1]
    per_w = R // _SC_WORKERS
    nchunk = per_w // _GCHUNK
    mesh = plsc.VectorSubcoreMesh(core_axis_name="c", subcore_axis_name="s")

    @functools.partial(
        pl.kernel, mesh=mesh,
        out_type=jax.ShapeDtypeStruct((R, C), jnp.float32),
        scratch_types=[
            pltpu.VMEM((_GCHUNK,), jnp.int32),
            pltpu.VMEM((_GCHUNK, C), jnp.float32),
            pltpu.SemaphoreType.DMA,
        ],
    )
    def gather_k(table_hbm, idx_hbm, out_hbm, idx_v, rows_v, sem):
        wid = lax.axis_index("s") * _SC_CORES + lax.axis_index("c")
        wbase = wid * per_w
        for ch in range(nchunk):
            base = wbase + ch * _GCHUNK
            pltpu.sync_copy(idx_hbm.at[pl.ds(base, _GCHUNK)], idx_v)
            pltpu.async_copy(table_hbm.at[idx_v], rows_v, sem).wait()
            pltpu.sync_copy(rows_v, out_hbm.at[pl.ds(base, _GCHUNK)])

    return gather_k(table, idx)


def _conv_body(feat_ref, xt_ref, w_ref, pre_ref):
    fr = feat_ref[0]
    xr = xt_ref[0]
    C = xr.shape[1]
    w_bf = w_ref[...].astype(jnp.bfloat16)
    xr_bf = xr.astype(jnp.bfloat16)
    acc = jnp.zeros((w_bf.shape[0], xr.shape[0]), jnp.float32)
    for t in range(K):
        nbr = fr[:, t * C:(t + 1) * C]
        ff = jnp.concatenate([(nbr - xr).astype(jnp.bfloat16), xr_bf],
                             axis=1)
        acc = acc + lax.dot_general(w_bf, ff, (((1,), (1,)), ((), ())),
                                    preferred_element_type=jnp.float32)
    pre_ref[0] = acc / float(K)


def _conv(feat, xt, w, *, nblk=8):
    B, N, _ = feat.shape
    C = xt.shape[2]
    O = w.shape[0]
    bs = N // nblk
    return pl.pallas_call(
        _conv_body,
        grid=(B, nblk),
        in_specs=[
            pl.BlockSpec((1, bs, KPAD * C), lambda b, r: (b, r, 0)),
            pl.BlockSpec((1, bs, C), lambda b, r: (b, r, 0)),
            pl.BlockSpec((O, 2 * C), lambda b, r: (0, 0)),
        ],
        out_specs=pl.BlockSpec((1, O, bs), lambda b, r: (b, 0, r)),
        out_shape=jax.ShapeDtypeStruct((B, O, N), jnp.float32),
    )(feat, xt, w)


def _gcn_body(pre_ref, res_ref, out_ref):
    p = pre_ref[0]
    N = p.shape[1]
    mu = jnp.mean(p, axis=1, keepdims=True)
    dev = p - mu
    var = jnp.sum(dev * dev, axis=1, keepdims=True) / (N - 1)
    y = dev / jnp.sqrt(var + 0.001)
    y = jnp.maximum(y, 0.0)
    if res_ref is not None:
        y = y + res_ref[0]
    out_ref[0] = y


def _gcn(pre, residual):
    B, O, N = pre.shape
    if residual is None:
        def kern(pre_ref, out_ref):
            _gcn_body(pre_ref, None, out_ref)
        operands = (pre,)
        in_specs = [pl.BlockSpec((1, O, N), lambda b: (b, 0, 0))]
    else:
        kern = _gcn_body
        operands = (pre, residual)
        in_specs = [pl.BlockSpec((1, O, N), lambda b: (b, 0, 0)),
                    pl.BlockSpec((1, O, N), lambda b: (b, 0, 0))]
    return pl.pallas_call(
        kern,
        grid=(B,),
        in_specs=in_specs,
        out_specs=pl.BlockSpec((1, O, N), lambda b: (b, 0, 0)),
        out_shape=jax.ShapeDtypeStruct((B, O, N), jnp.float32),
    )(*operands)


def _layer(x, xt, w, residual):
    B, C, N = x.shape
    idx = _topk(x)
    feat = _sc_gather(xt.reshape(B * N, C), idx.reshape(B * N * KPAD))
    feat = feat.reshape(B, N, KPAD * C)
    pre = _conv(feat, xt, w)
    return _gcn(pre, residual)


def kernel(x, W1, b1, W2, b2):
    del b1, b2
    xt = jnp.transpose(x, (0, 2, 1))
    h = _layer(x, xt, W1, None)
    ht = jnp.transpose(h, (0, 2, 1))
    return _layer(h, ht, W2, x)


# speedup vs baseline: 8.1468x; 1.1778x over previous
import functools

import jax
import jax.numpy as jnp
from jax import lax
from jax.experimental import pallas as pl
from jax.experimental.pallas import tpu as pltpu
from jax.experimental.pallas import tpu_sc as plsc

K = 10
KPAD = 10
_NEG_INF = float("-inf")

_SC_CORES = 2
_SC_SUBCORES = 16
_SC_WORKERS = _SC_CORES * _SC_SUBCORES
_GCHUNK = 128


def _topk_body(x_ref, idx_ref, *, nblk):
    x = x_ref[0]
    C, N = x.shape
    bs = N // nblk
    base = pl.program_id(0) * N
    sq = jnp.sum(x * x, axis=0, keepdims=True)
    x_bf = x.astype(jnp.bfloat16)
    for r in range(nblk):
        xb = x[:, r * bs:(r + 1) * bs]
        g = lax.dot_general(xb.astype(jnp.bfloat16), x_bf,
                            (((0,), (0,)), ((), ())),
                            preferred_element_type=jnp.float32)
        d = 2.0 * g - sq
        iot = lax.broadcasted_iota(jnp.int32, (bs, N), 1)
        cs = []
        for _ in range(K):
            mx = jnp.max(d, axis=1, keepdims=True)
            eq = d == mx
            c = jnp.min(jnp.where(eq, iot, N), axis=1, keepdims=True)
            cs.append(c)
            d = jnp.where(iot == c, _NEG_INF, d)
        idxmat = jnp.concatenate(cs, axis=1)
        idx_ref[0, r * bs:(r + 1) * bs, :] = idxmat + base


def _topk(x, *, nblk=8):
    B, C, N = x.shape
    return pl.pallas_call(
        functools.partial(_topk_body, nblk=nblk),
        grid=(B,),
        in_specs=[pl.BlockSpec((1, C, N), lambda b: (b, 0, 0))],
        out_specs=pl.BlockSpec((1, N, KPAD), lambda b: (b, 0, 0)),
        out_shape=jax.ShapeDtypeStruct((B, N, KPAD), jnp.int32),
    )(x)


def _sc_gather(table, idx):
    R = idx.shape[0]
    C = table.shape[1]
    per_w = R // _SC_WORKERS
    nchunk = per_w // _GCHUNK
    mesh = plsc.VectorSubcoreMesh(core_axis_name="c", subcore_axis_name="s")

    @functools.partial(
        pl.kernel, mesh=mesh,
        out_type=jax.ShapeDtypeStruct((R, C), jnp.float32),
        scratch_types=[
            pltpu.VMEM((_GCHUNK,), jnp.int32),
            pltpu.VMEM((_GCHUNK, C), jnp.float32),
            pltpu.SemaphoreType.DMA,
        ],
    )
    def gather_k(table_hbm, idx_hbm, out_hbm, idx_v, rows_v, sem):
        wid = lax.axis_index("s") * _SC_CORES + lax.axis_index("c")
        wbase = wid * per_w
        for ch in range(nchunk):
            base = wbase + ch * _GCHUNK
            pltpu.sync_copy(idx_hbm.at[pl.ds(base, _GCHUNK)], idx_v)
            pltpu.async_copy(table_hbm.at[idx_v], rows_v, sem).wait()
            pltpu.sync_copy(rows_v, out_hbm.at[pl.ds(base, _GCHUNK)])

    return gather_k(table, idx)


def _conv_body(feat_ref, xt_ref, w_ref, pre_ref):
    fr = feat_ref[0]
    xr = xt_ref[0]
    C = xr.shape[1]
    w_bf = w_ref[...].astype(jnp.bfloat16)
    xr_bf = xr.astype(jnp.bfloat16)
    acc = jnp.zeros((w_bf.shape[0], xr.shape[0]), jnp.float32)
    for t in range(K):
        nbr = fr[:, t * C:(t + 1) * C]
        ff = jnp.concatenate([(nbr - xr).astype(jnp.bfloat16), xr_bf],
                             axis=1)
        acc = acc + lax.dot_general(w_bf, ff, (((1,), (1,)), ((), ())),
                                    preferred_element_type=jnp.float32)
    pre_ref[0] = acc / float(K)


def _conv(feat, xt, w, *, nblk=8):
    B, N, _ = feat.shape
    C = xt.shape[2]
    O = w.shape[0]
    bs = N // nblk
    return pl.pallas_call(
        _conv_body,
        grid=(B, nblk),
        in_specs=[
            pl.BlockSpec((1, bs, KPAD * C), lambda b, r: (b, r, 0)),
            pl.BlockSpec((1, bs, C), lambda b, r: (b, r, 0)),
            pl.BlockSpec((O, 2 * C), lambda b, r: (0, 0)),
        ],
        out_specs=pl.BlockSpec((1, O, bs), lambda b, r: (b, 0, r)),
        out_shape=jax.ShapeDtypeStruct((B, O, N), jnp.float32),
    )(feat, xt, w)


def _gcn_body(pre_ref, res_ref, out_ref):
    p = pre_ref[0]
    N = p.shape[1]
    mu = jnp.mean(p, axis=1, keepdims=True)
    dev = p - mu
    var = jnp.sum(dev * dev, axis=1, keepdims=True) / (N - 1)
    y = dev / jnp.sqrt(var + 0.001)
    y = jnp.maximum(y, 0.0)
    if res_ref is not None:
        y = y + res_ref[0]
    out_ref[0] = y


def _gcn(pre, residual):
    B, O, N = pre.shape
    if residual is None:
        def kern(pre_ref, out_ref):
            _gcn_body(pre_ref, None, out_ref)
        operands = (pre,)
        in_specs = [pl.BlockSpec((1, O, N), lambda b: (b, 0, 0))]
    else:
        kern = _gcn_body
        operands = (pre, residual)
        in_specs = [pl.BlockSpec((1, O, N), lambda b: (b, 0, 0)),
                    pl.BlockSpec((1, O, N), lambda b: (b, 0, 0))]
    return pl.pallas_call(
        kern,
        grid=(B,),
        in_specs=in_specs,
        out_specs=pl.BlockSpec((1, O, N), lambda b: (b, 0, 0)),
        out_shape=jax.ShapeDtypeStruct((B, O, N), jnp.float32),
    )(*operands)


def _layer(x, xt, w, residual):
    B, C, N = x.shape
    idx = _topk(x)
    feat = _sc_gather(xt.reshape(B * N, C), idx.reshape(B * N * KPAD))
    feat = feat.reshape(B, N, KPAD * C)
    pre = _conv(feat, xt, w)
    return _gcn(pre, residual)


def kernel(x, W1, b1, W2, b2):
    del b1, b2
    xt = jnp.transpose(x, (0, 2, 1))
    h = _layer(x, xt, W1, None)
    ht = jnp.transpose(h, (0, 2, 1))
    return _layer(h, ht, W2, x)


# speedup vs baseline: 8.3006x; 1.0189x over previous
import functools

import jax
import jax.numpy as jnp
from jax import lax
from jax.experimental import pallas as pl
from jax.experimental.pallas import tpu as pltpu
from jax.experimental.pallas import tpu_sc as plsc

K = 10
KPAD = 10
_NEG_INF = float("-inf")

_SC_CORES = 2
_SC_SUBCORES = 16
_SC_WORKERS = _SC_CORES * _SC_SUBCORES
_GCHUNK = 128


def _topk_body(x_ref, idx_ref, *, nblk):
    x = x_ref[0]
    C, N = x.shape
    bs = N // nblk
    base = pl.program_id(0) * N
    sq = jnp.sum(x * x, axis=0, keepdims=True)
    x_bf = x.astype(jnp.bfloat16)
    for r in range(nblk):
        xb = x[:, r * bs:(r + 1) * bs]
        g = lax.dot_general(xb.astype(jnp.bfloat16), x_bf,
                            (((0,), (0,)), ((), ())),
                            preferred_element_type=jnp.float32)
        d = 2.0 * g - sq
        iot = lax.broadcasted_iota(jnp.int32, (bs, N), 1)
        cs = []
        for _ in range(K):
            mx = jnp.max(d, axis=1, keepdims=True)
            eq = d == mx
            c = jnp.min(jnp.where(eq, iot, N), axis=1, keepdims=True)
            cs.append(c)
            d = jnp.where(iot == c, _NEG_INF, d)
        idxmat = jnp.concatenate(cs, axis=1)
        idx_ref[0, r * bs:(r + 1) * bs, :] = idxmat + base


def _topk(x, *, nblk=8):
    B, C, N = x.shape
    return pl.pallas_call(
        functools.partial(_topk_body, nblk=nblk),
        grid=(B,),
        in_specs=[pl.BlockSpec((1, C, N), lambda b: (b, 0, 0))],
        out_specs=pl.BlockSpec((1, N, KPAD), lambda b: (b, 0, 0)),
        out_shape=jax.ShapeDtypeStruct((B, N, KPAD), jnp.int32),
    )(x)


def _sc_gather(table, idx):
    R = idx.shape[0]
    C = table.shape[1]
    per_w = R // _SC_WORKERS
    nchunk = per_w // _GCHUNK
    mesh = plsc.VectorSubcoreMesh(core_axis_name="c", subcore_axis_name="s")

    @functools.partial(
        pl.kernel, mesh=mesh,
        out_type=jax.ShapeDtypeStruct((R, C), jnp.float32),
        scratch_types=[
            pltpu.VMEM((_GCHUNK,), jnp.int32),
            pltpu.VMEM((_GCHUNK,), jnp.int32),
            pltpu.VMEM((_GCHUNK, C), jnp.float32),
            pltpu.VMEM((_GCHUNK, C), jnp.float32),
            pltpu.SemaphoreType.DMA,
            pltpu.SemaphoreType.DMA,
        ],
    )
    def gather_k(table_hbm, idx_hbm, out_hbm, i0, i1, r0, r1, s0, s1):
        wid = lax.axis_index("s") * _SC_CORES + lax.axis_index("c")
        wbase = wid * per_w
        bufs = [(i0, r0, s0), (i1, r1, s1)]
        handles = [None, None]
        iv, rv, sv = bufs[0]
        pltpu.sync_copy(idx_hbm.at[pl.ds(wbase, _GCHUNK)], iv)
        handles[0] = pltpu.async_copy(table_hbm.at[iv], rv, sv)
        for ch in range(nchunk):
            cur = ch % 2
            nxt = (ch + 1) % 2
            if ch + 1 < nchunk:
                iv, rv, sv = bufs[nxt]
                base = wbase + (ch + 1) * _GCHUNK
                pltpu.sync_copy(idx_hbm.at[pl.ds(base, _GCHUNK)], iv)
                handles[nxt] = pltpu.async_copy(table_hbm.at[iv], rv, sv)
            handles[cur].wait()
            pltpu.sync_copy(bufs[cur][1],
                            out_hbm.at[pl.ds(wbase + ch * _GCHUNK, _GCHUNK)])

    return gather_k(table, idx)


def _conv_body(feat_ref, xt_ref, w_ref, pre_ref):
    fr = feat_ref[0]
    xr = xt_ref[0]
    C = xr.shape[1]
    w_bf = w_ref[...].astype(jnp.bfloat16)
    xr_bf = xr.astype(jnp.bfloat16)
    acc = jnp.zeros((w_bf.shape[0], xr.shape[0]), jnp.float32)
    for t in range(K):
        nbr = fr[:, t * C:(t + 1) * C]
        ff = jnp.concatenate([(nbr - xr).astype(jnp.bfloat16), xr_bf],
                             axis=1)
        acc = acc + lax.dot_general(w_bf, ff, (((1,), (1,)), ((), ())),
                                    preferred_element_type=jnp.float32)
    pre_ref[0] = acc / float(K)


def _conv(feat, xt, w, *, nblk=8):
    B, N, _ = feat.shape
    C = xt.shape[2]
    O = w.shape[0]
    bs = N // nblk
    return pl.pallas_call(
        _conv_body,
        grid=(B, nblk),
        in_specs=[
            pl.BlockSpec((1, bs, KPAD * C), lambda b, r: (b, r, 0)),
            pl.BlockSpec((1, bs, C), lambda b, r: (b, r, 0)),
            pl.BlockSpec((O, 2 * C), lambda b, r: (0, 0)),
        ],
        out_specs=pl.BlockSpec((1, O, bs), lambda b, r: (b, 0, r)),
        out_shape=jax.ShapeDtypeStruct((B, O, N), jnp.float32),
    )(feat, xt, w)


def _gcn_body(pre_ref, res_ref, out_ref):
    p = pre_ref[0]
    N = p.shape[1]
    mu = jnp.mean(p, axis=1, keepdims=True)
    dev = p - mu
    var = jnp.sum(dev * dev, axis=1, keepdims=True) / (N - 1)
    y = dev / jnp.sqrt(var + 0.001)
    y = jnp.maximum(y, 0.0)
    if res_ref is not None:
        y = y + res_ref[0]
    out_ref[0] = y


def _gcn(pre, residual):
    B, O, N = pre.shape
    if residual is None:
        def kern(pre_ref, out_ref):
            _gcn_body(pre_ref, None, out_ref)
        operands = (pre,)
        in_specs = [pl.BlockSpec((1, O, N), lambda b: (b, 0, 0))]
    else:
        kern = _gcn_body
        operands = (pre, residual)
        in_specs = [pl.BlockSpec((1, O, N), lambda b: (b, 0, 0)),
                    pl.BlockSpec((1, O, N), lambda b: (b, 0, 0))]
    return pl.pallas_call(
        kern,
        grid=(B,),
        in_specs=in_specs,
        out_specs=pl.BlockSpec((1, O, N), lambda b: (b, 0, 0)),
        out_shape=jax.ShapeDtypeStruct((B, O, N), jnp.float32),
    )(*operands)


def _layer(x, xt, w, residual):
    B, C, N = x.shape
    idx = _topk(x)
    feat = _sc_gather(xt.reshape(B * N, C), idx.reshape(B * N * KPAD))
    feat = feat.reshape(B, N, KPAD * C)
    pre = _conv(feat, xt, w)
    return _gcn(pre, residual)


def kernel(x, W1, b1, W2, b2):
    del b1, b2
    xt = jnp.transpose(x, (0, 2, 1))
    h = _layer(x, xt, W1, None)
    ht = jnp.transpose(h, (0, 2, 1))
    return _layer(h, ht, W2, x)


# speedup vs baseline: 8.3993x; 1.0119x over previous
import functools

import jax
import jax.numpy as jnp
from jax import lax
from jax.experimental import pallas as pl
from jax.experimental.pallas import tpu as pltpu
from jax.experimental.pallas import tpu_sc as plsc

K = 10
KPAD = 10
_NEG_INF = float("-inf")

_SC_CORES = 2
_SC_SUBCORES = 16
_SC_WORKERS = _SC_CORES * _SC_SUBCORES
_GCHUNK = 128


def _topk_body(x_ref, idx_ref, *, nblk):
    x = x_ref[0]
    C, N = x.shape
    bs = N // nblk
    base = pl.program_id(0) * N
    sq = jnp.sum(x * x, axis=0, keepdims=True)
    x_bf = x.astype(jnp.bfloat16)
    for r in range(nblk):
        xb = x[:, r * bs:(r + 1) * bs]
        g = lax.dot_general(xb.astype(jnp.bfloat16), x_bf,
                            (((0,), (0,)), ((), ())),
                            preferred_element_type=jnp.float32)
        d = 2.0 * g - sq
        iot = lax.broadcasted_iota(jnp.int32, (bs, N), 1)
        cs = []
        for _ in range(K):
            mx = jnp.max(d, axis=1, keepdims=True)
            eq = d == mx
            c = jnp.min(jnp.where(eq, iot, N), axis=1, keepdims=True)
            cs.append(c)
            d = jnp.where(iot == c, _NEG_INF, d)
        idxmat = jnp.concatenate(cs, axis=1)
        idx_ref[0, r * bs:(r + 1) * bs, :] = idxmat + base


def _topk(x, *, nblk=4):
    B, C, N = x.shape
    return pl.pallas_call(
        functools.partial(_topk_body, nblk=nblk),
        grid=(B,),
        in_specs=[pl.BlockSpec((1, C, N), lambda b: (b, 0, 0))],
        out_specs=pl.BlockSpec((1, N, KPAD), lambda b: (b, 0, 0)),
        out_shape=jax.ShapeDtypeStruct((B, N, KPAD), jnp.int32),
    )(x)


def _sc_gather(table, idx):
    R = idx.shape[0]
    C = table.shape[1]
    per_w = R // _SC_WORKERS
    nchunk = per_w // _GCHUNK
    mesh = plsc.VectorSubcoreMesh(core_axis_name="c", subcore_axis_name="s")

    @functools.partial(
        pl.kernel, mesh=mesh,
        out_type=jax.ShapeDtypeStruct((R, C), jnp.float32),
        scratch_types=[
            pltpu.VMEM((_GCHUNK,), jnp.int32),
            pltpu.VMEM((_GCHUNK,), jnp.int32),
            pltpu.VMEM((_GCHUNK, C), jnp.float32),
            pltpu.VMEM((_GCHUNK, C), jnp.float32),
            pltpu.SemaphoreType.DMA,
            pltpu.SemaphoreType.DMA,
        ],
    )
    def gather_k(table_hbm, idx_hbm, out_hbm, i0, i1, r0, r1, s0, s1):
        wid = lax.axis_index("s") * _SC_CORES + lax.axis_index("c")
        wbase = wid * per_w
        bufs = [(i0, r0, s0), (i1, r1, s1)]
        handles = [None, None]
        iv, rv, sv = bufs[0]
        pltpu.sync_copy(idx_hbm.at[pl.ds(wbase, _GCHUNK)], iv)
        handles[0] = pltpu.async_copy(table_hbm.at[iv], rv, sv)
        for ch in range(nchunk):
            cur = ch % 2
            nxt = (ch + 1) % 2
            if ch + 1 < nchunk:
                iv, rv, sv = bufs[nxt]
                base = wbase + (ch + 1) * _GCHUNK
                pltpu.sync_copy(idx_hbm.at[pl.ds(base, _GCHUNK)], iv)
                handles[nxt] = pltpu.async_copy(table_hbm.at[iv], rv, sv)
            handles[cur].wait()
            pltpu.sync_copy(bufs[cur][1],
                            out_hbm.at[pl.ds(wbase + ch * _GCHUNK, _GCHUNK)])

    return gather_k(table, idx)


def _conv_body(feat_ref, xt_ref, w_ref, pre_ref):
    fr = feat_ref[0]
    xr = xt_ref[0]
    C = xr.shape[1]
    w_bf = w_ref[...].astype(jnp.bfloat16)
    xr_bf = xr.astype(jnp.bfloat16)
    acc = jnp.zeros((w_bf.shape[0], xr.shape[0]), jnp.float32)
    for t in range(K):
        nbr = fr[:, t * C:(t + 1) * C]
        ff = jnp.concatenate([(nbr - xr).astype(jnp.bfloat16), xr_bf],
                             axis=1)
        acc = acc + lax.dot_general(w_bf, ff, (((1,), (1,)), ((), ())),
                                    preferred_element_type=jnp.float32)
    pre_ref[0] = acc / float(K)


def _conv(feat, xt, w, *, nblk=8):
    B, N, _ = feat.shape
    C = xt.shape[2]
    O = w.shape[0]
    bs = N // nblk
    return pl.pallas_call(
        _conv_body,
        grid=(B, nblk),
        in_specs=[
            pl.BlockSpec((1, bs, KPAD * C), lambda b, r: (b, r, 0)),
            pl.BlockSpec((1, bs, C), lambda b, r: (b, r, 0)),
            pl.BlockSpec((O, 2 * C), lambda b, r: (0, 0)),
        ],
        out_specs=pl.BlockSpec((1, O, bs), lambda b, r: (b, 0, r)),
        out_shape=jax.ShapeDtypeStruct((B, O, N), jnp.float32),
    )(feat, xt, w)


def _gcn_body(pre_ref, res_ref, out_ref):
    p = pre_ref[0]
    N = p.shape[1]
    mu = jnp.mean(p, axis=1, keepdims=True)
    dev = p - mu
    var = jnp.sum(dev * dev, axis=1, keepdims=True) / (N - 1)
    y = dev / jnp.sqrt(var + 0.001)
    y = jnp.maximum(y, 0.0)
    if res_ref is not None:
        y = y + res_ref[0]
    out_ref[0] = y


def _gcn(pre, residual):
    B, O, N = pre.shape
    if residual is None:
        def kern(pre_ref, out_ref):
            _gcn_body(pre_ref, None, out_ref)
        operands = (pre,)
        in_specs = [pl.BlockSpec((1, O, N), lambda b: (b, 0, 0))]
    else:
        kern = _gcn_body
        operands = (pre, residual)
        in_specs = [pl.BlockSpec((1, O, N), lambda b: (b, 0, 0)),
                    pl.BlockSpec((1, O, N), lambda b: (b, 0, 0))]
    return pl.pallas_call(
        kern,
        grid=(B,),
        in_specs=in_specs,
        out_specs=pl.BlockSpec((1, O, N), lambda b: (b, 0, 0)),
        out_shape=jax.ShapeDtypeStruct((B, O, N), jnp.float32),
    )(*operands)


def _layer(x, xt, w, residual):
    B, C, N = x.shape
    idx = _topk(x)
    feat = _sc_gather(xt.reshape(B * N, C), idx.reshape(B * N * KPAD))
    feat = feat.reshape(B, N, KPAD * C)
    pre = _conv(feat, xt, w)
    return _gcn(pre, residual)


def kernel(x, W1, b1, W2, b2):
    del b1, b2
    xt = jnp.transpose(x, (0, 2, 1))
    h = _layer(x, xt, W1, None)
    ht = jnp.transpose(h, (0, 2, 1))
    return _layer(h, ht, W2, x)


# speedup vs baseline: 8.4224x; 1.0028x over previous
import functools

import jax
import jax.numpy as jnp
from jax import lax
from jax.experimental import pallas as pl
from jax.experimental.pallas import tpu as pltpu
from jax.experimental.pallas import tpu_sc as plsc

K = 10
KPAD = 10
_NEG_INF = float("-inf")

_SC_CORES = 2
_SC_SUBCORES = 16
_SC_WORKERS = _SC_CORES * _SC_SUBCORES
_GCHUNK = 128


def _topk_body(x_ref, idx_ref, *, nblk):
    x = x_ref[0]
    C, N = x.shape
    bs = N // nblk
    base = pl.program_id(0) * N
    sq = jnp.sum(x * x, axis=0, keepdims=True)
    x_bf = x.astype(jnp.bfloat16)
    for r in range(nblk):
        xb = x[:, r * bs:(r + 1) * bs]
        g = lax.dot_general(xb.astype(jnp.bfloat16), x_bf,
                            (((0,), (0,)), ((), ())),
                            preferred_element_type=jnp.float32)
        d = 2.0 * g - sq
        iot = lax.broadcasted_iota(jnp.int32, (bs, N), 1)
        cs = []
        for _ in range(K):
            mx = jnp.max(d, axis=1, keepdims=True)
            eq = d == mx
            c = jnp.min(jnp.where(eq, iot, N), axis=1, keepdims=True)
            cs.append(c)
            d = jnp.where(iot == c, _NEG_INF, d)
        idxmat = jnp.concatenate(cs, axis=1)
        idx_ref[0, r * bs:(r + 1) * bs, :] = idxmat + base


def _topk(x, *, nblk=2):
    B, C, N = x.shape
    return pl.pallas_call(
        functools.partial(_topk_body, nblk=nblk),
        grid=(B,),
        in_specs=[pl.BlockSpec((1, C, N), lambda b: (b, 0, 0))],
        out_specs=pl.BlockSpec((1, N, KPAD), lambda b: (b, 0, 0)),
        out_shape=jax.ShapeDtypeStruct((B, N, KPAD), jnp.int32),
    )(x)


def _sc_gather(table, idx):
    R = idx.shape[0]
    C = table.shape[1]
    per_w = R // _SC_WORKERS
    nchunk = per_w // _GCHUNK
    mesh = plsc.VectorSubcoreMesh(core_axis_name="c", subcore_axis_name="s")

    @functools.partial(
        pl.kernel, mesh=mesh,
        out_type=jax.ShapeDtypeStruct((R, C), jnp.float32),
        scratch_types=[
            pltpu.VMEM((_GCHUNK,), jnp.int32),
            pltpu.VMEM((_GCHUNK,), jnp.int32),
            pltpu.VMEM((_GCHUNK, C), jnp.float32),
            pltpu.VMEM((_GCHUNK, C), jnp.float32),
            pltpu.SemaphoreType.DMA,
            pltpu.SemaphoreType.DMA,
        ],
    )
    def gather_k(table_hbm, idx_hbm, out_hbm, i0, i1, r0, r1, s0, s1):
        wid = lax.axis_index("s") * _SC_CORES + lax.axis_index("c")
        wbase = wid * per_w
        bufs = [(i0, r0, s0), (i1, r1, s1)]
        handles = [None, None]
        iv, rv, sv = bufs[0]
        pltpu.sync_copy(idx_hbm.at[pl.ds(wbase, _GCHUNK)], iv)
        handles[0] = pltpu.async_copy(table_hbm.at[iv], rv, sv)
        for ch in range(nchunk):
            cur = ch % 2
            nxt = (ch + 1) % 2
            if ch + 1 < nchunk:
                iv, rv, sv = bufs[nxt]
                base = wbase + (ch + 1) * _GCHUNK
                pltpu.sync_copy(idx_hbm.at[pl.ds(base, _GCHUNK)], iv)
                handles[nxt] = pltpu.async_copy(table_hbm.at[iv], rv, sv)
            handles[cur].wait()
            pltpu.sync_copy(bufs[cur][1],
                            out_hbm.at[pl.ds(wbase + ch * _GCHUNK, _GCHUNK)])

    return gather_k(table, idx)


def _conv_body(feat_ref, xt_ref, w_ref, pre_ref):
    fr = feat_ref[0]
    xr = xt_ref[0]
    C = xr.shape[1]
    w_bf = w_ref[...].astype(jnp.bfloat16)
    xr_bf = xr.astype(jnp.bfloat16)
    acc = jnp.zeros((w_bf.shape[0], xr.shape[0]), jnp.float32)
    for t in range(K):
        nbr = fr[:, t * C:(t + 1) * C]
        ff = jnp.concatenate([(nbr - xr).astype(jnp.bfloat16), xr_bf],
                             axis=1)
        acc = acc + lax.dot_general(w_bf, ff, (((1,), (1,)), ((), ())),
                                    preferred_element_type=jnp.float32)
    pre_ref[0] = acc / float(K)


def _conv(feat, xt, w, *, nblk=8):
    B, N, _ = feat.shape
    C = xt.shape[2]
    O = w.shape[0]
    bs = N // nblk
    return pl.pallas_call(
        _conv_body,
        grid=(B, nblk),
        in_specs=[
            pl.BlockSpec((1, bs, KPAD * C), lambda b, r: (b, r, 0)),
            pl.BlockSpec((1, bs, C), lambda b, r: (b, r, 0)),
            pl.BlockSpec((O, 2 * C), lambda b, r: (0, 0)),
        ],
        out_specs=pl.BlockSpec((1, O, bs), lambda b, r: (b, 0, r)),
        out_shape=jax.ShapeDtypeStruct((B, O, N), jnp.float32),
    )(feat, xt, w)


def _gcn_body(pre_ref, res_ref, out_ref):
    p = pre_ref[0]
    N = p.shape[1]
    mu = jnp.mean(p, axis=1, keepdims=True)
    dev = p - mu
    var = jnp.sum(dev * dev, axis=1, keepdims=True) / (N - 1)
    y = dev / jnp.sqrt(var + 0.001)
    y = jnp.maximum(y, 0.0)
    if res_ref is not None:
        y = y + res_ref[0]
    out_ref[0] = y


def _gcn(pre, residual):
    B, O, N = pre.shape
    if residual is None:
        def kern(pre_ref, out_ref):
            _gcn_body(pre_ref, None, out_ref)
        operands = (pre,)
        in_specs = [pl.BlockSpec((1, O, N), lambda b: (b, 0, 0))]
    else:
        kern = _gcn_body
        operands = (pre, residual)
        in_specs = [pl.BlockSpec((1, O, N), lambda b: (b, 0, 0)),
                    pl.BlockSpec((1, O, N), lambda b: (b, 0, 0))]
    return pl.pallas_call(
        kern,
        grid=(B,),
        in_specs=in_specs,
        out_specs=pl.BlockSpec((1, O, N), lambda b: (b, 0, 0)),
        out_shape=jax.ShapeDtypeStruct((B, O, N), jnp.float32),
    )(*operands)


def _layer(x, xt, w, residual):
    B, C, N = x.shape
    idx = _topk(x)
    feat = _sc_gather(xt.reshape(B * N, C), idx.reshape(B * N * KPAD))
    feat = feat.reshape(B, N, KPAD * C)
    pre = _conv(feat, xt, w)
    return _gcn(pre, residual)


def kernel(x, W1, b1, W2, b2):
    del b1, b2
    xt = jnp.transpose(x, (0, 2, 1))
    h = _layer(x, xt, W1, None)
    ht = jnp.transpose(h, (0, 2, 1))
    return _layer(h, ht, W2, x)


# speedup vs baseline: 10.2339x; 1.2151x over previous
import functools

import jax
import jax.numpy as jnp
from jax import lax
from jax.experimental import pallas as pl
from jax.experimental.pallas import tpu as pltpu
from jax.experimental.pallas import tpu_sc as plsc

K = 10
KPAD = 10
_NEG_INF = float("-inf")

_SC_CORES = 2
_SC_SUBCORES = 16
_SC_WORKERS = _SC_CORES * _SC_SUBCORES
_GCHUNK = 128


def _topk_body(x_ref, idx_ref, *, nblk):
    x = x_ref[0]
    C, N = x.shape
    bs = N // nblk
    base = pl.program_id(0) * N
    sq = jnp.sum(x * x, axis=0, keepdims=True)
    x_bf = x.astype(jnp.bfloat16)
    for r in range(nblk):
        xb = x[:, r * bs:(r + 1) * bs]
        g = lax.dot_general(xb.astype(jnp.bfloat16), x_bf,
                            (((0,), (0,)), ((), ())),
                            preferred_element_type=jnp.float32)
        d = 2.0 * g - sq
        iot = lax.broadcasted_iota(jnp.int32, (bs, N), 1)
        cs = []
        for _ in range(K):
            mx = jnp.max(d, axis=1, keepdims=True)
            eq = d == mx
            c = jnp.min(jnp.where(eq, iot, N), axis=1, keepdims=True)
            cs.append(c)
            d = jnp.where(iot == c, _NEG_INF, d)
        idxmat = jnp.concatenate(cs, axis=1)
        idx_ref[0, r * bs:(r + 1) * bs, :] = idxmat + base


def _topk(x, *, nblk=2):
    B, C, N = x.shape
    return pl.pallas_call(
        functools.partial(_topk_body, nblk=nblk),
        grid=(B,),
        in_specs=[pl.BlockSpec((1, C, N), lambda b: (b, 0, 0))],
        out_specs=pl.BlockSpec((1, N, KPAD), lambda b: (b, 0, 0)),
        out_shape=jax.ShapeDtypeStruct((B, N, KPAD), jnp.int32),
    )(x)


def _sc_gather(table, idx):
    R = idx.shape[0]
    C = table.shape[1]
    per_w = R // _SC_WORKERS
    nchunk = per_w // _GCHUNK
    mesh = plsc.VectorSubcoreMesh(core_axis_name="c", subcore_axis_name="s")

    @functools.partial(
        pl.kernel, mesh=mesh,
        out_type=jax.ShapeDtypeStruct((R, C), jnp.float32),
        scratch_types=[
            pltpu.VMEM((_GCHUNK,), jnp.int32),
            pltpu.VMEM((_GCHUNK,), jnp.int32),
            pltpu.VMEM((_GCHUNK, C), jnp.float32),
            pltpu.VMEM((_GCHUNK, C), jnp.float32),
            pltpu.SemaphoreType.DMA,
            pltpu.SemaphoreType.DMA,
        ],
    )
    def gather_k(table_hbm, idx_hbm, out_hbm, i0, i1, r0, r1, s0, s1):
        wid = lax.axis_index("s") * _SC_CORES + lax.axis_index("c")
        wbase = wid * per_w
        bufs = [(i0, r0, s0), (i1, r1, s1)]
        handles = [None, None]
        iv, rv, sv = bufs[0]
        pltpu.sync_copy(idx_hbm.at[pl.ds(wbase, _GCHUNK)], iv)
        handles[0] = pltpu.async_copy(table_hbm.at[iv], rv, sv)
        for ch in range(nchunk):
            cur = ch % 2
            nxt = (ch + 1) % 2
            if ch + 1 < nchunk:
                iv, rv, sv = bufs[nxt]
                base = wbase + (ch + 1) * _GCHUNK
                pltpu.sync_copy(idx_hbm.at[pl.ds(base, _GCHUNK)], iv)
                handles[nxt] = pltpu.async_copy(table_hbm.at[iv], rv, sv)
            handles[cur].wait()
            pltpu.sync_copy(bufs[cur][1],
                            out_hbm.at[pl.ds(wbase + ch * _GCHUNK, _GCHUNK)])

    return gather_k(table, idx)


def _conv_body(feat_ref, xt_ref, w_ref, pre_ref):
    fr = feat_ref[0]
    xr = xt_ref[0]
    C = xr.shape[1]
    w_bf = w_ref[...].astype(jnp.bfloat16)
    xr_bf = xr.astype(jnp.bfloat16)
    acc = jnp.zeros((w_bf.shape[0], xr.shape[0]), jnp.float32)
    for t in range(K):
        nbr = fr[:, t * C:(t + 1) * C]
        ff = jnp.concatenate([(nbr - xr).astype(jnp.bfloat16), xr_bf],
                             axis=1)
        acc = acc + lax.dot_general(w_bf, ff, (((1,), (1,)), ((), ())),
                                    preferred_element_type=jnp.float32)
    pre_ref[0] = acc / float(K)


def _conv(feat, xt, w, *, nblk=8):
    B, N, _ = feat.shape
    C = xt.shape[2]
    O = w.shape[0]
    bs = N // nblk
    return pl.pallas_call(
        _conv_body,
        grid=(B, nblk),
        in_specs=[
            pl.BlockSpec((1, bs, KPAD * C), lambda b, r: (b, r, 0)),
            pl.BlockSpec((1, bs, C), lambda b, r: (b, r, 0)),
            pl.BlockSpec((O, 2 * C), lambda b, r: (0, 0)),
        ],
        out_specs=pl.BlockSpec((1, O, bs), lambda b, r: (b, 0, r)),
        out_shape=jax.ShapeDtypeStruct((B, O, N), jnp.float32),
    )(feat, xt, w)


def _gcn_body(pre_ref, res_ref, out_ref):
    p = pre_ref[0]
    N = p.shape[1]
    mu = jnp.mean(p, axis=1, keepdims=True)
    dev = p - mu
    var = jnp.sum(dev * dev, axis=1, keepdims=True) / (N - 1)
    y = dev / jnp.sqrt(var + 0.001)
    y = jnp.maximum(y, 0.0)
    if res_ref is not None:
        y = y + res_ref[0]
    out_ref[0] = y


def _gcn(pre, residual):
    B, O, N = pre.shape
    if residual is None:
        def kern(pre_ref, out_ref):
            _gcn_body(pre_ref, None, out_ref)
        operands = (pre,)
        in_specs = [pl.BlockSpec((1, O, N), lambda b: (b, 0, 0))]
    else:
        kern = _gcn_body
        operands = (pre, residual)
        in_specs = [pl.BlockSpec((1, O, N), lambda b: (b, 0, 0)),
                    pl.BlockSpec((1, O, N), lambda b: (b, 0, 0))]
    return pl.pallas_call(
        kern,
        grid=(B,),
        in_specs=in_specs,
        out_specs=pl.BlockSpec((1, O, N), lambda b: (b, 0, 0)),
        out_shape=jax.ShapeDtypeStruct((B, O, N), jnp.float32),
    )(*operands)


def _layer(x, xt, w, residual):
    B, C, N = x.shape
    table = xt.reshape(B * N, C)
    idxs = [_topk(x[b:b + 1]) for b in range(B)]
    feats = [_sc_gather(table, idxs[b].reshape(N * KPAD) + b * N)
             for b in range(B)]
    outs = []
    for b in range(B):
        pre = _conv(feats[b].reshape(1, N, KPAD * C), xt[b:b + 1], w)
        outs.append(_gcn(pre, None if residual is None
                         else residual[b:b + 1]))
    return jnp.concatenate(outs, axis=0)


def kernel(x, W1, b1, W2, b2):
    del b1, b2
    xt = jnp.transpose(x, (0, 2, 1))
    h = _layer(x, xt, W1, None)
    ht = jnp.transpose(h, (0, 2, 1))
    return _layer(h, ht, W2, x)
